# baseline (speedup 1.0000x reference)
import functools
import jax
import jax.numpy as jnp
from jax import lax
from jax.experimental import pallas as pl
from jax.experimental.pallas import tpu as pltpu
from jax.experimental.pallas import tpu_sc as plsc

_N = 8192
_K = 4096
_B = 128
_ROWS = 64
_NS = 16
_CHUNK = _N // _NS
_NV = _CHUNK // 16


def _popcnt(b):
    return plsc.all_reduce_population_count(b)


def _sc_mask_kernel(w_hbm, mask_hbm, uv, pv, zv, maskv, shared):
    c = lax.axis_index("c")
    s = lax.axis_index("s")

    @pl.when(c == 0)
    def _run():
        iota = lax.iota(jnp.int32, 16)
        base_off = s * _CHUNK

        pltpu.sync_copy(w_hbm.at[pl.ds(base_off, _CHUNK)], uv)
        zv[...] = jnp.zeros((16,), jnp.int32)
        pltpu.sync_copy(zv, shared.at[pl.ds(s * 16, 16)])

        def _stage(i, carry):
            wi = uv[pl.ds(i * 16, 16)]
            uv[pl.ds(i * 16, 16)] = wi & jnp.int32(0x7FFFFFFF)
            return carry
        lax.fori_loop(0, _NV, _stage, jnp.int32(0))
        plsc.subcore_barrier()

        kvec = jnp.full((16,), _K, jnp.int32)

        t = jnp.zeros((16,), jnp.int32)
        for r, (b, hi) in enumerate(((28, 7), (24, 15), (20, 15), (16, 15),
                                     (12, 15), (8, 15), (4, 15), (0, 15))):
            cands = [t | jnp.int32(j << b) for j in range(1, hi + 1)]

            def _cnt(i, accs):
                ui = uv[pl.ds(i * 16, 16)]
                return tuple(acc + _popcnt(ui >= cands[j])
                             for j, acc in enumerate(accs))
            accs = lax.fori_loop(
                0, _NV, _cnt,
                tuple(jnp.zeros((16,), jnp.int32) for _ in range(hi)))

            pvv = jnp.zeros((16,), jnp.int32)
            for j in range(hi):
                pvv = jnp.where(iota == j, accs[j], pvv)
            pv[...] = pvv
            pltpu.sync_copy(pv, shared.at[iota + 16 * r], add=True)
            plsc.subcore_barrier()
            pltpu.sync_copy(shared.at[pl.ds(16 * r, 16)], pv)
            d = _popcnt(pv[...] >= kvec)
            t = t | lax.shift_left(d, b)

        def _gt(i, acc):
            ui = uv[pl.ds(i * 16, 16)]
            return acc + _popcnt(ui > t)
        gta = lax.fori_loop(0, _NV, _gt, jnp.zeros((16,), jnp.int32))
        pv[...] = gta
        pltpu.sync_copy(pv, shared.at[iota + 128], add=True)
        plsc.subcore_barrier()
        pltpu.sync_copy(shared.at[pl.ds(128, 16)], pv)
        ties_m1 = jnp.full((16,), _K - 1, jnp.int32) - pv[...]

        fbase = base_off + iota
        m = jnp.zeros((16,), jnp.int32)
        for r, b in enumerate((12, 8, 4, 0)):
            cands = [m | jnp.int32(j << b) for j in range(1, 16)]

            def _icnt(i, accs):
                ui = uv[pl.ds(i * 16, 16)]
                eqv = ui == t
                fidx = fbase + i * 16
                return tuple(acc + _popcnt(eqv & (fidx < cands[j]))
                             for j, acc in enumerate(accs))
            accs = lax.fori_loop(
                0, _NV, _icnt,
                tuple(jnp.zeros((16,), jnp.int32) for _ in range(15)))

            pvv = jnp.zeros((16,), jnp.int32)
            for j in range(15):
                pvv = jnp.where(iota == j, accs[j], pvv)
            pv[...] = pvv
            pltpu.sync_copy(pv, shared.at[iota + 144 + 16 * r], add=True)
            plsc.subcore_barrier()
            pltpu.sync_copy(shared.at[pl.ds(144 + 16 * r, 16)], pv)
            d = _popcnt((pv[...] <= ties_m1) & (iota < 15))
            m = m | lax.shift_left(d, b)

        def _mask(i, carry):
            ui = uv[pl.ds(i * 16, 16)]
            fidx = fbase + i * 16
            keep = (ui > t) | ((ui == t) & (fidx <= m))
            maskv[pl.ds(i * 16, 16)] = jnp.where(keep, jnp.float32(1.0),
                                                 jnp.float32(0.0))
            return carry
        lax.fori_loop(0, _NV, _mask, jnp.int32(0))
        pltpu.sync_copy(maskv, mask_hbm.at[pl.ds(base_off, _CHUNK)])


_sc_mask = functools.partial(
    pl.kernel,
    out_type=jax.ShapeDtypeStruct((_N,), jnp.float32),
    mesh=plsc.VectorSubcoreMesh(core_axis_name="c", subcore_axis_name="s",
                                num_cores=2, num_subcores=16),
    compiler_params=pltpu.CompilerParams(needs_layout_passes=False),
    scratch_types=[
        pltpu.VMEM((_CHUNK,), jnp.int32),
        pltpu.VMEM((16,), jnp.int32),
        pltpu.VMEM((16,), jnp.int32),
        pltpu.VMEM((_CHUNK,), jnp.float32),
        pltpu.VMEM_SHARED((256,), jnp.int32),
    ],
)(_sc_mask_kernel)


def _mul_body(x_ref, m_ref, o_ref):
    o_ref[...] = x_ref[...] * m_ref[...]


def kernel(x, weights):
    w_bits = lax.bitcast_convert_type(weights, jnp.int32)
    mask = _sc_mask(w_bits)
    sel = pl.pallas_call(
        _mul_body,
        grid=(_B // _ROWS,),
        in_specs=[
            pl.BlockSpec((_ROWS, _N), lambda i: (i, 0)),
            pl.BlockSpec((1, _N), lambda i: (0, 0)),
        ],
        out_specs=pl.BlockSpec((_ROWS, _N), lambda i: (i, 0)),
        out_shape=jax.ShapeDtypeStruct((_B, _N), jnp.float32),
    )(x, mask.reshape(1, _N))
    return (sel, mask)


# speedup vs baseline: 4.5679x; 4.5679x over previous
import jax
import jax.numpy as jnp
from jax.experimental import pallas as pl

_N = 8192
_K = 4096
_B = 128
_R = 64
_C = 128
_ROWS = 64


def _vsum(ind):
    s = jnp.sum(jnp.where(ind, jnp.int32(1), jnp.int32(0)), axis=0,
                keepdims=True)
    return jnp.sum(s, axis=1, keepdims=True)


def _mask_body(w_ref, mask_ref):
    v = jnp.abs(w_ref[...])
    u = jax.lax.bitcast_convert_type(v, jnp.int32)

    t = jnp.zeros((1, 1), jnp.int32)
    for b, hi in ((28, 7), (24, 15), (20, 15), (16, 15),
                  (12, 15), (8, 15), (4, 15), (0, 15)):
        d = jnp.zeros((1, 1), jnp.int32)
        for j in range(1, hi + 1):
            cnt = _vsum(u >= (t | jnp.int32(j << b)))
            d = d + jnp.where(cnt >= _K, jnp.int32(1), jnp.int32(0))
        t = t | jax.lax.shift_left(d, b)

    gt = u > t
    eq = u == t
    ties = (_K - _vsum(gt)).astype(jnp.float32)

    eqf = jnp.where(eq, jnp.float32(1.0), jnp.float32(0.0))
    jj = jax.lax.broadcasted_iota(jnp.int32, (_C, _C), 0)
    cc = jax.lax.broadcasted_iota(jnp.int32, (_C, _C), 1)
    tri_c = jnp.where(jj < cc, jnp.float32(1.0), jnp.float32(0.0))
    inrow = jnp.dot(eqf, tri_c, preferred_element_type=jnp.float32)
    rowsum = jnp.sum(eqf, axis=1, keepdims=True)
    r0 = jax.lax.broadcasted_iota(jnp.int32, (_R, _R), 0)
    r1 = jax.lax.broadcasted_iota(jnp.int32, (_R, _R), 1)
    tri_r = jnp.where(r1 < r0, jnp.float32(1.0), jnp.float32(0.0))
    rowpre = jnp.dot(tri_r, rowsum, preferred_element_type=jnp.float32)
    prefix = inrow + rowpre

    keep = gt | (eq & (prefix < ties))
    mask_ref[...] = jnp.where(keep, jnp.float32(1.0), jnp.float32(0.0))


def _mul_body(x_ref, m_ref, o_ref):
    o_ref[...] = x_ref[...] * m_ref[...]


def kernel(x, weights):
    w2 = weights.reshape(_R, _C)
    mask2 = pl.pallas_call(
        _mask_body,
        out_shape=jax.ShapeDtypeStruct((_R, _C), jnp.float32),
    )(w2)
    mask = mask2.reshape(_N)

    sel = pl.pallas_call(
        _mul_body,
        grid=(_B // _ROWS,),
        in_specs=[
            pl.BlockSpec((_ROWS, _N), lambda i: (i, 0)),
            pl.BlockSpec((1, _N), lambda i: (0, 0)),
        ],
        out_specs=pl.BlockSpec((_ROWS, _N), lambda i: (i, 0)),
        out_shape=jax.ShapeDtypeStruct((_B, _N), jnp.float32),
    )(x, mask.reshape(1, _N))
    return (sel, mask)
